# baseline (speedup 1.0000x reference)
import functools

import jax
import jax.numpy as jnp
from jax.experimental import pallas as pl
from jax.experimental.pallas import tpu as pltpu


def _conv_body(x_ref, w_ref, b_ref, scale_ref, shift_ref,
               y_ref, ps_ref, sq_ref, *, apply_act, W):
    B, Cin = x_ref.shape[0], x_ref.shape[1]
    HW = y_ref.shape[2]

    col = jax.lax.broadcasted_iota(jnp.int32, (1, B * HW), 1) % W
    not_first = col != 0
    not_last = col != W - 1
    zpad = jnp.zeros((Cin, W + 1), jnp.bfloat16)

    if apply_act:
        scale, shift = scale_ref[...], shift_ref[...]

    xps = []
    for i in range(B):
        x = x_ref[i]
        if x.ndim == 3:
            x = x.reshape(Cin, HW)
        if apply_act:
            z = x.astype(jnp.float32) * scale + shift
            x = jnp.where(z >= 0, z, 0.01 * z)
        xb = x.astype(jnp.bfloat16)
        xps.append(jnp.concatenate([zpad, xb, zpad], axis=1))

    rows = []
    for kh in range(3):
        for kw in range(3):
            s = kh * W + kw
            t = jnp.concatenate(
                [jax.lax.slice_in_dim(xp, s, s + HW, axis=1) for xp in xps],
                axis=1)
            if kw == 0:
                t = jnp.where(not_first, t, jnp.bfloat16(0))
            elif kw == 2:
                t = jnp.where(not_last, t, jnp.bfloat16(0))
            rows.append(t)
    patch = jnp.concatenate(rows, axis=0)

    acc = jnp.dot(w_ref[...], patch,
                  preferred_element_type=jnp.float32)
    acc = acc + b_ref[...]

    for i in range(B):
        a = jax.lax.slice_in_dim(acc, i * HW, (i + 1) * HW, axis=1)
        ps_ref[i] = jnp.sum(a, axis=1, keepdims=True)
        sq_ref[i] = jnp.sum(a * a, axis=1, keepdims=True)
        y_ref[i] = a.astype(y_ref.dtype)


def _conv3x3(xin, wt, b, scale, shift, *, apply_act, W, nb):
    N, Cin = xin.shape[0], xin.shape[1]
    HW = xin.shape[2] if xin.ndim == 3 else xin.shape[2] * xin.shape[3]
    Cout = wt.shape[0]
    if xin.ndim == 3:
        x_spec = pl.BlockSpec((nb, Cin, HW), lambda n: (n, 0, 0))
    else:
        x_spec = pl.BlockSpec((nb, Cin, xin.shape[2], xin.shape[3]),
                              lambda n: (n, 0, 0, 0))
    body = functools.partial(_conv_body, apply_act=apply_act, W=W)
    return pl.pallas_call(
        body,
        out_shape=(
            jax.ShapeDtypeStruct((N, Cout, HW), jnp.bfloat16),
            jax.ShapeDtypeStruct((N, Cout, 1), jnp.float32),
            jax.ShapeDtypeStruct((N, Cout, 1), jnp.float32),
        ),
        grid_spec=pltpu.PrefetchScalarGridSpec(
            num_scalar_prefetch=0,
            grid=(N // nb,),
            in_specs=[
                x_spec,
                pl.BlockSpec((Cout, 9 * Cin), lambda n: (0, 0)),
                pl.BlockSpec((Cout, 1), lambda n: (0, 0)),
                pl.BlockSpec((Cin, 1), lambda n: (0, 0)),
                pl.BlockSpec((Cin, 1), lambda n: (0, 0)),
            ],
            out_specs=[
                pl.BlockSpec((nb, Cout, HW), lambda n: (n, 0, 0)),
                pl.BlockSpec((nb, Cout, 1), lambda n: (n, 0, 0)),
                pl.BlockSpec((nb, Cout, 1), lambda n: (n, 0, 0)),
            ],
        ),
        compiler_params=pltpu.CompilerParams(
            dimension_semantics=("parallel",)),
    )(xin, wt, b, scale, shift)


def _affine_body(y_ref, scale_ref, shift_ref, o_ref):
    z = y_ref[...].astype(jnp.float32) * scale_ref[...] + shift_ref[...]
    z = jnp.where(z >= 0, z, 0.01 * z)
    o_ref[...] = z.reshape(o_ref.shape)


def _affine_lrelu(y, scale, shift, *, nb, H, W):
    N, C, HW = y.shape
    return pl.pallas_call(
        _affine_body,
        out_shape=jax.ShapeDtypeStruct((N, C, H, W), jnp.float32),
        grid_spec=pltpu.PrefetchScalarGridSpec(
            num_scalar_prefetch=0,
            grid=(N // nb,),
            in_specs=[
                pl.BlockSpec((nb, C, HW), lambda i: (i, 0, 0)),
                pl.BlockSpec((C, 1), lambda i: (0, 0)),
                pl.BlockSpec((C, 1), lambda i: (0, 0)),
            ],
            out_specs=pl.BlockSpec((nb, C, H, W), lambda i: (i, 0, 0, 0)),
        ),
        compiler_params=pltpu.CompilerParams(
            dimension_semantics=("parallel",)),
    )(y, scale, shift)


def kernel(x, w1, b1, gamma1, beta1, w2, b2, gamma2, beta2):
    N, Cin, H, W = x.shape
    HW = H * W
    C1 = w1.shape[-1]
    C2 = w2.shape[-1]

    w1t = w1.reshape(9 * Cin, C1).T.astype(jnp.bfloat16)
    w2t = w2.reshape(9 * C1, C2).T.astype(jnp.bfloat16)
    b1c = b1.reshape(C1, 1).astype(jnp.float32)
    b2c = b2.reshape(C2, 1).astype(jnp.float32)
    dummy = jnp.zeros((Cin, 1), jnp.float32)

    y1, ps1, sq1 = _conv3x3(x, w1t, b1c, dummy, dummy,
                            apply_act=False, W=W, nb=8)
    scale1, shift1 = _bn_fold(ps1, sq1, gamma1, beta1, N * HW)
    y2, ps2, sq2 = _conv3x3(y1, w2t, b2c, scale1, shift1,
                            apply_act=True, W=W, nb=2)
    scale2, shift2 = _bn_fold(ps2, sq2, gamma2, beta2, N * HW)
    return _affine_lrelu(y2, scale2, shift2, nb=8, H=H, W=W)


def _bn_fold(ps, sq, gamma, beta, count, eps=1e-5):
    s = jnp.sum(ps[:, :, 0], axis=0)
    q = jnp.sum(sq[:, :, 0], axis=0)
    mean = s / count
    var = jnp.maximum(q / count - mean * mean, 0.0)
    scale = gamma[0] / jnp.sqrt(var + eps)
    shift = beta[0] - mean * scale
    return scale.reshape(-1, 1), shift.reshape(-1, 1)


# speedup vs baseline: 1.0949x; 1.0949x over previous
import functools

import jax
import jax.numpy as jnp
from jax.experimental import pallas as pl
from jax.experimental.pallas import tpu as pltpu


def _conv_body(x_ref, w_ref, b_ref, scale_ref, shift_ref,
               y_ref, ps_ref, sq_ref, *, apply_act, W):
    B, Cin = x_ref.shape[0], x_ref.shape[1]
    HW = y_ref.shape[2]

    col = jax.lax.broadcasted_iota(jnp.int32, (1, B * HW), 1) % W
    not_first = col != 0
    not_last = col != W - 1
    zpad = jnp.zeros((Cin, W + 1), jnp.bfloat16)

    if apply_act:
        scale, shift = scale_ref[...], shift_ref[...]

    xps = []
    for i in range(B):
        x = x_ref[i]
        if x.ndim == 3:
            x = x.reshape(Cin, HW)
        if apply_act:
            z = x.astype(jnp.float32) * scale + shift
            x = jnp.where(z >= 0, z, 0.01 * z)
        xb = x.astype(jnp.bfloat16)
        xps.append(jnp.concatenate([zpad, xb, zpad], axis=1))

    rows = []
    for kh in range(3):
        for kw in range(3):
            s = kh * W + kw
            t = jnp.concatenate(
                [jax.lax.slice_in_dim(xp, s, s + HW, axis=1) for xp in xps],
                axis=1)
            if kw == 0:
                t = jnp.where(not_first, t, jnp.bfloat16(0))
            elif kw == 2:
                t = jnp.where(not_last, t, jnp.bfloat16(0))
            rows.append(t)
    patch = jnp.concatenate(rows, axis=0)

    acc = jnp.dot(w_ref[...], patch,
                  preferred_element_type=jnp.float32)
    acc = acc + b_ref[...]

    for i in range(B):
        a = jax.lax.slice_in_dim(acc, i * HW, (i + 1) * HW, axis=1)
        ps_ref[i] = jnp.sum(a, axis=1, keepdims=True)
        sq_ref[i] = jnp.sum(a * a, axis=1, keepdims=True)
        y_ref[i] = a.astype(y_ref.dtype)


def _conv3x3(xin, wt, b, scale, shift, *, apply_act, W, nb):
    N, Cin = xin.shape[0], xin.shape[1]
    HW = xin.shape[2] if xin.ndim == 3 else xin.shape[2] * xin.shape[3]
    Cout = wt.shape[0]
    if xin.ndim == 3:
        x_spec = pl.BlockSpec((nb, Cin, HW), lambda n: (n, 0, 0))
    else:
        x_spec = pl.BlockSpec((nb, Cin, xin.shape[2], xin.shape[3]),
                              lambda n: (n, 0, 0, 0))
    body = functools.partial(_conv_body, apply_act=apply_act, W=W)
    return pl.pallas_call(
        body,
        out_shape=(
            jax.ShapeDtypeStruct((N, Cout, HW), jnp.bfloat16),
            jax.ShapeDtypeStruct((N, Cout, 1), jnp.float32),
            jax.ShapeDtypeStruct((N, Cout, 1), jnp.float32),
        ),
        grid_spec=pltpu.PrefetchScalarGridSpec(
            num_scalar_prefetch=0,
            grid=(N // nb,),
            in_specs=[
                x_spec,
                pl.BlockSpec((Cout, 9 * Cin), lambda n: (0, 0)),
                pl.BlockSpec((Cout, 1), lambda n: (0, 0)),
                pl.BlockSpec((Cin, 1), lambda n: (0, 0)),
                pl.BlockSpec((Cin, 1), lambda n: (0, 0)),
            ],
            out_specs=[
                pl.BlockSpec((nb, Cout, HW), lambda n: (n, 0, 0)),
                pl.BlockSpec((nb, Cout, 1), lambda n: (n, 0, 0)),
                pl.BlockSpec((nb, Cout, 1), lambda n: (n, 0, 0)),
            ],
        ),
        compiler_params=pltpu.CompilerParams(
            dimension_semantics=("parallel",)),
    )(xin, wt, b, scale, shift)


def _affine_body(y_ref, scale_ref, shift_ref, o_ref):
    z = y_ref[...].astype(jnp.float32) * scale_ref[...] + shift_ref[...]
    z = jnp.where(z >= 0, z, 0.01 * z)
    o_ref[...] = z.reshape(o_ref.shape)


def _affine_lrelu(y, scale, shift, *, nb, H, W):
    N, C, HW = y.shape
    return pl.pallas_call(
        _affine_body,
        out_shape=jax.ShapeDtypeStruct((N, C, H, W), jnp.float32),
        grid_spec=pltpu.PrefetchScalarGridSpec(
            num_scalar_prefetch=0,
            grid=(N // nb,),
            in_specs=[
                pl.BlockSpec((nb, C, HW), lambda i: (i, 0, 0)),
                pl.BlockSpec((C, 1), lambda i: (0, 0)),
                pl.BlockSpec((C, 1), lambda i: (0, 0)),
            ],
            out_specs=pl.BlockSpec((nb, C, H, W), lambda i: (i, 0, 0, 0)),
        ),
        compiler_params=pltpu.CompilerParams(
            dimension_semantics=("parallel",)),
    )(y, scale, shift)


def kernel(x, w1, b1, gamma1, beta1, w2, b2, gamma2, beta2):
    N, Cin, H, W = x.shape
    HW = H * W
    C1 = w1.shape[-1]
    C2 = w2.shape[-1]

    w1t = w1.reshape(9 * Cin, C1).T.astype(jnp.bfloat16)
    w2t = w2.reshape(9 * C1, C2).T.astype(jnp.bfloat16)
    b1c = b1.reshape(C1, 1).astype(jnp.float32)
    b2c = b2.reshape(C2, 1).astype(jnp.float32)
    dummy = jnp.zeros((Cin, 1), jnp.float32)

    y1, ps1, sq1 = _conv3x3(x, w1t, b1c, dummy, dummy,
                            apply_act=False, W=W, nb=8)
    scale1, shift1 = _bn_fold(ps1, sq1, gamma1, beta1, N * HW)
    y2, ps2, sq2 = _conv3x3(y1, w2t, b2c, scale1, shift1,
                            apply_act=True, W=W, nb=8)
    scale2, shift2 = _bn_fold(ps2, sq2, gamma2, beta2, N * HW)
    return _affine_lrelu(y2, scale2, shift2, nb=8, H=H, W=W)


def _bn_fold(ps, sq, gamma, beta, count, eps=1e-5):
    s = jnp.sum(ps[:, :, 0], axis=0)
    q = jnp.sum(sq[:, :, 0], axis=0)
    mean = s / count
    var = jnp.maximum(q / count - mean * mean, 0.0)
    scale = gamma[0] / jnp.sqrt(var + eps)
    shift = beta[0] - mean * scale
    return scale.reshape(-1, 1), shift.reshape(-1, 1)
